# baseline (speedup 1.0000x reference)
import functools

import jax
import jax.numpy as jnp
from jax import lax
from jax.experimental import pallas as pl
from jax.experimental.pallas import tpu as pltpu
from jax.experimental.pallas import tpu_sc as plsc

POOL_H, POOL_W = 7, 7
NUM_LEVELS = 4
C = 256
CW = C // 2
L = 16
NPIX = POOL_H * POOL_W
NPIXP = 64
NIDX = NPIX * 4
NIDP = 200
CH0, CH1 = 104, 96
NWP = 200
NW = 32


def _precompute(boxes, box_fpn_level, level_hw):
    B, N = boxes.shape[0], boxes.shape[1]
    M = B * N
    flat_boxes = boxes.reshape(M, 4)
    flat_lvl = box_fpn_level.reshape(-1)
    flat_batch = jnp.repeat(jnp.arange(B, dtype=jnp.int32), N)

    sort_key = flat_batch * (NUM_LEVELS + 1) + flat_lvl
    perm = jnp.argsort(sort_key, stable=True)
    sb = flat_boxes[perm]
    slvl = flat_lvl[perm]
    sbatch = flat_batch[perm]

    hw = jnp.asarray(level_hw, dtype=jnp.int32)
    Hm = hw[slvl - 1]
    Hf = (Hm - 1).astype(jnp.float32)

    y1, x1, y2, x2 = sb[:, 0], sb[:, 1], sb[:, 2], sb[:, 3]
    i = jnp.arange(POOL_H, dtype=jnp.float32)
    j = jnp.arange(POOL_W, dtype=jnp.float32)
    in_y = y1[:, None] * Hf[:, None] + i[None, :] * ((y2 - y1) * Hf / (POOL_H - 1))[:, None]
    in_x = x1[:, None] * Hf[:, None] + j[None, :] * ((x2 - x1) * Hf / (POOL_W - 1))[:, None]
    y_ok = (in_y >= 0) & (in_y <= Hf[:, None])
    x_ok = (in_x >= 0) & (in_x <= Hf[:, None])
    y0f = jnp.floor(in_y)
    x0f = jnp.floor(in_x)
    wy = (in_y - y0f).astype(jnp.float32)
    wx = (in_x - x0f).astype(jnp.float32)
    y0 = jnp.clip(y0f, 0, Hf[:, None]).astype(jnp.int32)
    y1i = jnp.clip(y0f + 1, 0, Hf[:, None]).astype(jnp.int32)
    x0 = jnp.clip(x0f, 0, Hf[:, None]).astype(jnp.int32)
    x1i = jnp.clip(x0f + 1, 0, Hf[:, None]).astype(jnp.int32)

    base = (sbatch * Hm * Hm)[:, None, None]
    Wm = Hm[:, None, None]

    def rid(yy, xx):
        return base + yy[:, :, None] * Wm + xx[:, None, :]

    ids4 = jnp.stack([rid(y0, x0), rid(y0, x1i), rid(y1i, x0), rid(y1i, x1i)],
                     axis=-1)

    omy, omx = 1.0 - wy, 1.0 - wx
    w4 = jnp.stack([omy[:, :, None] * omx[:, None, :],
                    omy[:, :, None] * wx[:, None, :],
                    wy[:, :, None] * omx[:, None, :],
                    wy[:, :, None] * wx[:, None, :]], axis=-1)
    mask = (y_ok[:, :, None] & x_ok[:, None, :]).astype(jnp.float32)[..., None]
    w4 = w4 * mask

    return ids4.reshape(M, NIDX), w4.reshape(M, NIDX), slvl


def _make_sc_call(B, N, box_per_w):
    M = B * N
    mesh = plsc.VectorSubcoreMesh(core_axis_name="c", subcore_axis_name="s")

    @functools.partial(
        pl.kernel,
        mesh=mesh,
        compiler_params=pltpu.CompilerParams(needs_layout_passes=False),
        out_type=jax.ShapeDtypeStruct((M * NPIX, C), jnp.float32),
        scratch_types=[
            pltpu.VMEM((2 * NIDP,), jnp.int32),
            pltpu.VMEM((box_per_w + L,), jnp.int32),
            pltpu.VMEM((2 * NWP,), jnp.float32),
            pltpu.VMEM((2 * NIDP, CW), jnp.int32),
            pltpu.VMEM((NPIXP, C), jnp.float32),
            pltpu.SemaphoreType.DMA,
            pltpu.SemaphoreType.DMA,
            pltpu.SemaphoreType.DMA,
            pltpu.SemaphoreType.DMA,
        ],
    )
    def sc_call(t0, t1, t2, t3, ids_hbm, w_hbm, lvl_hbm, out_hbm,
                ids_v, lvl_v, w_v, rows_v, out_v, isem, gsem0, gsem1, osem):
        wid = lax.axis_index("s") * 2 + lax.axis_index("c")
        base = wid * box_per_w
        pltpu.sync_copy(lvl_hbm.at[pl.ds(base, box_per_w)],
                        lvl_v.at[pl.ds(0, box_per_w)])
        tables = (t0, t1, t2, t3)
        gsems = (gsem0, gsem1)
        chunks = ((0, CH0), (CH0, CH1))
        lane0 = lax.iota(jnp.int32, L) == 0
        iota = lax.iota(jnp.int32, L)

        def ids_copy(g, buf):
            return pltpu.make_async_copy(
                ids_hbm.at[pl.ds(g * NIDP, NIDP)],
                ids_v.at[pl.ds(buf * NIDP, NIDP)], isem)

        def w_copy(g, buf):
            return pltpu.make_async_copy(
                w_hbm.at[pl.ds(g * NWP, NWP)],
                w_v.at[pl.ds(buf * NWP, NWP)], gsems[buf])

        def fire_box(g, b, buf):
            lvl_vec = lvl_v[pl.ds(b, L)]
            w_copy(g, buf).start()
            preds = [jnp.any(lane0 & (lvl_vec == k + 1))
                     for k in range(NUM_LEVELS)]
            for k in range(NUM_LEVELS):
                @pl.when(preds[k])
                def _fire(tbl=tables[k], sm=gsems[buf]):
                    for off, sz in chunks:
                        pltpu.make_async_copy(
                            tbl.at[ids_v.at[pl.ds(buf * NIDP + off, sz)]],
                            rows_v.at[pl.ds(buf * NIDP + off, sz)], sm).start()

        def wait_box(g, buf):
            w_copy(g, buf).wait()
            for off, sz in chunks:
                pltpu.make_async_copy(
                    t0.at[ids_v.at[pl.ds(buf * NIDP + off, sz)]],
                    rows_v.at[pl.ds(buf * NIDP + off, sz)], gsems[buf]).wait()

        def drain_scatter():
            for o in (0, 16, 32, 48):
                pltpu.make_async_copy(
                    out_v.at[pl.ds(o, L)], out_hbm.at[iota], osem).wait()

        def blend_scatter(g, buf):
            rbase = buf * NIDP
            wbuf = buf * NWP

            @pl.when(g > base)
            def _drain_prev():
                drain_scatter()

            def pix_body(p, pcarry):
                r = 4 * p
                wb = wbuf + r

                def bcast(off):
                    return plsc.load_gather(
                        w_v, [jnp.full((L,), wb + off, jnp.int32)])

                w0, w1, w2, w3 = bcast(0), bcast(1), bcast(2), bcast(3)
                rr = rbase + r
                for gi in range(C // (2 * L)):
                    s2 = pl.ds(gi * L, L)

                    def up(k):
                        word = rows_v[rr + k, s2]
                        return plsc.unpack(
                            plsc.bitcast(word, jnp.bfloat16),
                            format=plsc.PackFormat.INTERLEAVED)

                    a0, b0 = up(0)
                    a1, b1 = up(1)
                    a2, b2 = up(2)
                    a3, b3 = up(3)
                    out_v[p, pl.ds(gi * 2 * L, L)] = (
                        a0 * w0 + a1 * w1 + a2 * w2 + a3 * w3)
                    out_v[p, pl.ds(gi * 2 * L + L, L)] = (
                        b0 * w0 + b1 * w1 + b2 * w2 + b3 * w3)
                return pcarry

            lax.fori_loop(0, NPIX, pix_body, 0, unroll=7)

            def dup_body(q, qcarry):
                for c in range(C // L):
                    s = pl.ds(c * L, L)
                    out_v[q, s] = out_v[NPIX - 1, s]
                return qcarry

            lax.fori_loop(NPIX, NPIXP, dup_body, 0)

            bb = g // N
            nn = g - bb * N
            c0 = bb * (NPIX * N) + nn
            for o in (0, 16, 32, 48):
                dvec = c0 + jnp.minimum(iota + o, NPIX - 1) * N
                pltpu.make_async_copy(
                    out_v.at[pl.ds(o, L)], out_hbm.at[dvec], osem).start()

        @pl.when(base < M)
        def _prologue():
            ids_copy(base, 0).start()
            ids_copy(base, 0).wait()
            fire_box(base, 0, 0)

            @pl.when(base + 1 < M)
            def _():
                ids_copy(base + 1, 1).start()

        def body(bh, carry):
            for par in (0, 1):
                b = 2 * bh + par
                g = base + b
                buf = par
                nbuf = 1 - par
                nb = b + 1
                gn = g + 1

                @pl.when((gn < M) & (nb < box_per_w))
                def _stage_next():
                    ids_copy(gn, nbuf).wait()
                    fire_box(gn, nb, nbuf)

                @pl.when(g < M)
                def _finish():
                    wait_box(g, buf)

                    @pl.when((gn + 1 < M) & (nb + 1 < box_per_w))
                    def _():
                        ids_copy(gn + 1, buf).start()

                    blend_scatter(g, buf)

            return carry

        lax.fori_loop(0, box_per_w // 2, body, 0)

        @pl.when(base < M)
        def _epilogue():
            drain_scatter()

    return sc_call


def kernel(boxes, feature_map_p2, feature_map_p3, feature_map_p4,
           feature_map_p5, box_fpn_level):
    feats = (feature_map_p2, feature_map_p3, feature_map_p4, feature_map_p5)
    B, N = boxes.shape[0], boxes.shape[1]
    M = B * N
    level_hw = tuple(f.shape[1] for f in feats)

    ids, w, slvl = _precompute(boxes, box_fpn_level, level_hw)

    box_per_w = -(-M // NW)
    box_per_w = -(-box_per_w // 8) * 8
    npad = box_per_w * NW
    pad = npad - M
    ids = jnp.pad(ids, ((0, pad), (0, NIDP - NIDX))).reshape(npad * NIDP)
    wpad = jnp.pad(w, ((0, 0), (0, NWP - NIDX))).reshape(M * NWP)
    slvl = jnp.pad(slvl, (0, pad), constant_values=1)

    def _to_bf16_packed(f, hw):
        v = f.reshape(B * hw * hw, C // 32, 2, L).astype(jnp.bfloat16)
        v = v.swapaxes(2, 3)
        return lax.bitcast_convert_type(v, jnp.int32).reshape(B * hw * hw, CW)

    tbls = tuple(_to_bf16_packed(f, hw) for f, hw in zip(feats, level_hw))
    sc_call = _make_sc_call(B, N, box_per_w)
    out = sc_call(*tbls, ids, wpad, slvl)
    return out.reshape(B, POOL_H, POOL_W, N, C).transpose(0, 3, 1, 2, 4)


# speedup vs baseline: 1.5408x; 1.5408x over previous
import functools

import jax
import jax.numpy as jnp
from jax import lax
from jax.experimental import pallas as pl
from jax.experimental.pallas import tpu as pltpu
from jax.experimental.pallas import tpu_sc as plsc

POOL_H, POOL_W = 7, 7
NUM_LEVELS = 4
C = 256
L = 16
NPIX = POOL_H * POOL_W
NPIXP = 64
NIDX = NPIX * 4
NIDP = 200
CH0, CH1 = 104, 96
NWP = 200
NW = 32


def _precompute(boxes, box_fpn_level, level_hw):
    B, N = boxes.shape[0], boxes.shape[1]
    M = B * N
    flat_boxes = boxes.reshape(M, 4)
    flat_lvl = box_fpn_level.reshape(-1)
    flat_batch = jnp.repeat(jnp.arange(B, dtype=jnp.int32), N)

    sort_key = flat_batch * (NUM_LEVELS + 1) + flat_lvl
    perm = jnp.argsort(sort_key, stable=True)
    sb = flat_boxes[perm]
    slvl = flat_lvl[perm]
    sbatch = flat_batch[perm]

    hw = jnp.asarray(level_hw, dtype=jnp.int32)
    Hm = hw[slvl - 1]
    Hf = (Hm - 1).astype(jnp.float32)

    y1, x1, y2, x2 = sb[:, 0], sb[:, 1], sb[:, 2], sb[:, 3]
    i = jnp.arange(POOL_H, dtype=jnp.float32)
    j = jnp.arange(POOL_W, dtype=jnp.float32)
    in_y = y1[:, None] * Hf[:, None] + i[None, :] * ((y2 - y1) * Hf / (POOL_H - 1))[:, None]
    in_x = x1[:, None] * Hf[:, None] + j[None, :] * ((x2 - x1) * Hf / (POOL_W - 1))[:, None]
    y_ok = (in_y >= 0) & (in_y <= Hf[:, None])
    x_ok = (in_x >= 0) & (in_x <= Hf[:, None])
    y0f = jnp.floor(in_y)
    x0f = jnp.floor(in_x)
    wy = (in_y - y0f).astype(jnp.float32)
    wx = (in_x - x0f).astype(jnp.float32)
    y0 = jnp.clip(y0f, 0, Hf[:, None]).astype(jnp.int32)
    y1i = jnp.clip(y0f + 1, 0, Hf[:, None]).astype(jnp.int32)
    x0 = jnp.clip(x0f, 0, Hf[:, None]).astype(jnp.int32)
    x1i = jnp.clip(x0f + 1, 0, Hf[:, None]).astype(jnp.int32)

    base = (sbatch * Hm * Hm)[:, None, None]
    Wm = Hm[:, None, None]

    def rid(yy, xx):
        return base + yy[:, :, None] * Wm + xx[:, None, :]

    ids4 = jnp.stack([rid(y0, x0), rid(y0, x1i), rid(y1i, x0), rid(y1i, x1i)],
                     axis=-1)

    omy, omx = 1.0 - wy, 1.0 - wx
    w4 = jnp.stack([omy[:, :, None] * omx[:, None, :],
                    omy[:, :, None] * wx[:, None, :],
                    wy[:, :, None] * omx[:, None, :],
                    wy[:, :, None] * wx[:, None, :]], axis=-1)
    mask = (y_ok[:, :, None] & x_ok[:, None, :]).astype(jnp.float32)[..., None]
    w4 = w4 * mask

    return ids4.reshape(M, NIDX), w4.reshape(M, NIDX), slvl


def _make_sc_call(B, N, box_per_w):
    M = B * N
    mesh = plsc.VectorSubcoreMesh(core_axis_name="c", subcore_axis_name="s")

    @functools.partial(
        pl.kernel,
        mesh=mesh,
        compiler_params=pltpu.CompilerParams(needs_layout_passes=False, use_tc_tiling_on_sc=True),
        out_type=jax.ShapeDtypeStruct((M * NPIX, C), jnp.float32),
        scratch_types=[
            pltpu.VMEM((2 * NIDP,), jnp.int32),
            pltpu.VMEM((box_per_w + L,), jnp.int32),
            pltpu.VMEM((2 * NWP,), jnp.float32),
            pltpu.VMEM((2 * NIDP, C), jnp.float32),
            pltpu.VMEM((NPIXP, C), jnp.float32),
            pltpu.SemaphoreType.DMA,
            pltpu.SemaphoreType.DMA,
            pltpu.SemaphoreType.DMA,
            pltpu.SemaphoreType.DMA,
        ],
    )
    def sc_call(t0, t1, t2, t3, ids_hbm, w_hbm, lvl_hbm, out_hbm,
                ids_v, lvl_v, w_v, rows_v, out_v, isem, gsem0, gsem1, osem):
        wid = lax.axis_index("s") * 2 + lax.axis_index("c")
        base = wid * box_per_w
        pltpu.sync_copy(lvl_hbm.at[pl.ds(base, box_per_w)],
                        lvl_v.at[pl.ds(0, box_per_w)])
        tables = (t0, t1, t2, t3)
        gsems = (gsem0, gsem1)
        chunks = ((0, CH0), (CH0, CH1))
        lane0 = lax.iota(jnp.int32, L) == 0
        iota = lax.iota(jnp.int32, L)

        def ids_copy(g, buf):
            return pltpu.make_async_copy(
                ids_hbm.at[pl.ds(g * NIDP, NIDP)],
                ids_v.at[pl.ds(buf * NIDP, NIDP)], isem)

        def w_copy(g, buf):
            return pltpu.make_async_copy(
                w_hbm.at[pl.ds(g * NWP, NWP)],
                w_v.at[pl.ds(buf * NWP, NWP)], gsems[buf])

        def fire_box(g, b, buf):
            lvl_vec = lvl_v[pl.ds(b, L)]
            w_copy(g, buf).start()
            preds = [jnp.any(lane0 & (lvl_vec == k + 1))
                     for k in range(NUM_LEVELS)]
            for k in range(NUM_LEVELS):
                @pl.when(preds[k])
                def _fire(tbl=tables[k], sm=gsems[buf]):
                    for off, sz in chunks:
                        pltpu.make_async_copy(
                            tbl.at[ids_v.at[pl.ds(buf * NIDP + off, sz)]],
                            rows_v.at[pl.ds(buf * NIDP + off, sz)], sm).start()

        def wait_box(g, buf):
            w_copy(g, buf).wait()
            for off, sz in chunks:
                pltpu.make_async_copy(
                    t0.at[ids_v.at[pl.ds(buf * NIDP + off, sz)]],
                    rows_v.at[pl.ds(buf * NIDP + off, sz)], gsems[buf]).wait()

        def drain_scatter():
            for o in (0, 16, 32, 48):
                pltpu.make_async_copy(
                    out_v.at[pl.ds(o, L)], out_hbm.at[iota], osem).wait()

        def blend_scatter(g, buf):
            rbase = buf * NIDP
            wbuf = buf * NWP

            @pl.when(g > base)
            def _drain_prev():
                drain_scatter()

            def pix_body(p, pcarry):
                r = 4 * p
                wb = wbuf + r

                def bcast(off):
                    return plsc.load_gather(
                        w_v, [jnp.full((L,), wb + off, jnp.int32)])

                w0, w1, w2, w3 = bcast(0), bcast(1), bcast(2), bcast(3)
                rr = rbase + r
                for c in range(C // L):
                    s = pl.ds(c * L, L)
                    out_v[p, s] = (rows_v[rr, s] * w0 + rows_v[rr + 1, s] * w1
                                   + rows_v[rr + 2, s] * w2
                                   + rows_v[rr + 3, s] * w3)
                return pcarry

            lax.fori_loop(0, NPIX, pix_body, 0, unroll=7)

            def dup_body(q, qcarry):
                for c in range(C // L):
                    s = pl.ds(c * L, L)
                    out_v[q, s] = out_v[NPIX - 1, s]
                return qcarry

            lax.fori_loop(NPIX, NPIXP, dup_body, 0)

            bb = g // N
            nn = g - bb * N
            c0 = bb * (NPIX * N) + nn
            for o in (0, 16, 32, 48):
                dvec = c0 + jnp.minimum(iota + o, NPIX - 1) * N
                pltpu.make_async_copy(
                    out_v.at[pl.ds(o, L)], out_hbm.at[dvec], osem).start()

        @pl.when(base < M)
        def _prologue():
            ids_copy(base, 0).start()
            ids_copy(base, 0).wait()
            fire_box(base, 0, 0)

            @pl.when(base + 1 < M)
            def _():
                ids_copy(base + 1, 1).start()

        def body(bh, carry):
            for par in (0, 1):
                b = 2 * bh + par
                g = base + b
                buf = par
                nbuf = 1 - par
                nb = b + 1
                gn = g + 1

                @pl.when((gn < M) & (nb < box_per_w))
                def _stage_next():
                    ids_copy(gn, nbuf).wait()
                    fire_box(gn, nb, nbuf)

                @pl.when(g < M)
                def _finish():
                    wait_box(g, buf)

                    @pl.when((gn + 1 < M) & (nb + 1 < box_per_w))
                    def _():
                        ids_copy(gn + 1, buf).start()

                    blend_scatter(g, buf)

            return carry

        lax.fori_loop(0, box_per_w // 2, body, 0)

        @pl.when(base < M)
        def _epilogue():
            drain_scatter()

    return sc_call


def kernel(boxes, feature_map_p2, feature_map_p3, feature_map_p4,
           feature_map_p5, box_fpn_level):
    feats = (feature_map_p2, feature_map_p3, feature_map_p4, feature_map_p5)
    B, N = boxes.shape[0], boxes.shape[1]
    M = B * N
    level_hw = tuple(f.shape[1] for f in feats)

    ids, w, slvl = _precompute(boxes, box_fpn_level, level_hw)

    box_per_w = -(-M // NW)
    box_per_w = -(-box_per_w // 8) * 8
    npad = box_per_w * NW
    pad = npad - M
    ids = jnp.pad(ids, ((0, pad), (0, NIDP - NIDX))).reshape(npad * NIDP)
    wpad = jnp.pad(w, ((0, 0), (0, NWP - NIDX))).reshape(M * NWP)
    slvl = jnp.pad(slvl, (0, pad), constant_values=1)

    tbls = tuple(f.reshape(B * hw * hw, C) for f, hw in zip(feats, level_hw))
    sc_call = _make_sc_call(B, N, box_per_w)
    out = sc_call(*tbls, ids, wpad, slvl)
    return out.reshape(B, POOL_H, POOL_W, N, C).transpose(0, 3, 1, 2, 4)
